# Initial kernel scaffold; baseline (speedup 1.0000x reference)
import jax
import jax.numpy as jnp
from jax.experimental import pallas as pl


def kernel(x, edge_index, batch, emb, W1, b1, W2, b2, Wc, bc):
    raise NotImplementedError("write your pallas kernel here")


# speedup vs baseline: 12.0305x; 12.0305x over previous
import functools

import jax
import jax.numpy as jnp
from jax import lax
from jax.experimental import pallas as pl
from jax.experimental.pallas import tpu as pltpu
from jax.experimental.pallas import tpu_sc as plsc

N = 100000
E = 3200000
D = 16
DH = D // 2
V = 50000
G = 128
LOUT = 10

NC = 2
NS = 16
NW = NC * NS

C = 128
NPAD = 100096
SLICE = NPAD // NS
NCH_E = E // C
NCH_N = N // C
NTAIL = N - NCH_N * C
ROWBLK = 800
NBLK = N // ROWBLK

_mesh = plsc.VectorSubcoreMesh(core_axis_name="c", subcore_axis_name="s")
_sc_params = pltpu.CompilerParams(use_tc_tiling_on_sc=False)


@functools.partial(
    pl.kernel,
    out_type=(
        jax.ShapeDtypeStruct((N, D), jnp.float32),
        jax.ShapeDtypeStruct((NC * NPAD,), jnp.float32),
    ),
    mesh=_mesh,
    scratch_types=[
        pltpu.VMEM((C,), jnp.int32),
        pltpu.VMEM((C, D), jnp.float32),
        pltpu.VMEM((NTAIL,), jnp.int32),
        pltpu.VMEM((NTAIL, D), jnp.float32),
        pltpu.VMEM((1, C), jnp.int32),
        pltpu.VMEM((C,), jnp.float32),
        pltpu.VMEM((SLICE,), jnp.float32),
        pltpu.VMEM_SHARED((NPAD,), jnp.float32),
        pltpu.SemaphoreType.DMA,
    ],
    compiler_params=_sc_params,
)
def _sc_embed_deg(x_hbm, emb_hbm, ei_hbm, zeros1_hbm, ones_hbm,
                  h0_hbm, degp_hbm,
                  xidx, rows, xidx_t, rows_t, didx, ones_v, zb, deg_sh, sem):
    c = lax.axis_index("c")
    s = lax.axis_index("s")
    wid = s * NC + c
    pltpu.sync_copy(zeros1_hbm.at[pl.ds(s * SLICE, SLICE)], zb)
    pltpu.sync_copy(zb, deg_sh.at[pl.ds(s * SLICE, SLICE)])
    pltpu.sync_copy(ones_hbm, ones_v)
    plsc.subcore_barrier()

    nte = 24 + (wid < NCH_N - 24 * NW).astype(jnp.int32)

    def ebody(t, carry):
        j = wid + NW * t
        base = j * C
        pltpu.sync_copy(x_hbm.at[pl.ds(base, C)], xidx)
        pltpu.async_copy(emb_hbm.at[xidx], rows, sem).wait()
        pltpu.sync_copy(rows, h0_hbm.at[pl.ds(base, C)])
        return carry

    lax.fori_loop(0, nte, ebody, 0)

    @pl.when(wid == 13)
    def _tail():
        pltpu.sync_copy(x_hbm.at[pl.ds(NCH_N * C, NTAIL)], xidx_t)
        pltpu.async_copy(emb_hbm.at[xidx_t], rows_t, sem).wait()
        pltpu.sync_copy(rows_t, h0_hbm.at[pl.ds(NCH_N * C, NTAIL)])

    ntd = 781 + (wid < NCH_E - 781 * NW).astype(jnp.int32)

    def dbody(t, carry):
        j = wid + NW * t
        base = j * C
        pltpu.sync_copy(ei_hbm.at[1, pl.ds(base, C)], didx.at[0])
        pltpu.sync_copy(ones_v, deg_sh.at[didx.at[0]], add=True)
        return carry

    lax.fori_loop(0, ntd, dbody, 0)
    plsc.subcore_barrier()
    pltpu.sync_copy(deg_sh.at[pl.ds(s * SLICE, SLICE)], zb)
    pltpu.sync_copy(zb, degp_hbm.at[pl.ds(c * NPAD + s * SLICE, SLICE)])


@functools.partial(
    pl.kernel,
    out_type=jax.ShapeDtypeStruct((NC, NPAD, DH), jnp.float32),
    mesh=_mesh,
    scratch_types=[
        pltpu.VMEM((C,), jnp.int32),
        pltpu.VMEM((1, C), jnp.int32),
        pltpu.VMEM((C, DH), jnp.float32),
        pltpu.VMEM((SLICE, DH), jnp.float32),
        pltpu.VMEM_SHARED((NPAD, DH), jnp.float32),
        pltpu.SemaphoreType.DMA,
    ],
    compiler_params=_sc_params,
)
def _sc_edge_agg(ei_hbm, glo_hbm, ghi_hbm, zeros2_hbm, aggp_hbm,
                 sidx, didx, rows, zb, agg_sh, sem):
    c = lax.axis_index("c")
    s = lax.axis_index("s")
    pltpu.sync_copy(zeros2_hbm.at[pl.ds(s * SLICE, SLICE)], zb)
    pltpu.sync_copy(zb, agg_sh.at[pl.ds(s * SLICE, SLICE)])
    plsc.subcore_barrier()

    nt = 1562 + (s < NCH_E - 1562 * NS).astype(jnp.int32)

    def sweep(g_hbm):
        def body(t, carry):
            j = s + NS * t
            base = j * C
            pltpu.sync_copy(ei_hbm.at[0, pl.ds(base, C)], sidx)
            pltpu.sync_copy(ei_hbm.at[1, pl.ds(base, C)], didx.at[0])
            pltpu.async_copy(g_hbm.at[sidx], rows, sem).wait()
            pltpu.sync_copy(rows, agg_sh.at[didx.at[0]], add=True)
            return carry

        lax.fori_loop(0, nt, body, 0)

    @pl.when(c == 0)
    def _lo():
        sweep(glo_hbm)

    @pl.when(c == 1)
    def _hi():
        sweep(ghi_hbm)

    plsc.subcore_barrier()
    pltpu.sync_copy(agg_sh.at[pl.ds(s * SLICE, SLICE)], zb)
    pltpu.sync_copy(zb, aggp_hbm.at[c, pl.ds(s * SLICE, SLICE)])


def _stage_a_body(degp_ref, h0_ref, w1_ref, glo_ref, ghi_ref, dinv_ref):
    deg = degp_ref[0] + degp_ref[1] + 1.0
    dinv = lax.rsqrt(deg)
    g1 = lax.dot_general(h0_ref[...], w1_ref[...],
                         (((1,), (1,)), ((), ())),
                         preferred_element_type=jnp.float32) * dinv
    glo_ref[...] = g1[:, :DH]
    ghi_ref[...] = g1[:, DH:]
    dinv_ref[...] = dinv


def _tc_stage_a(degp, h0, W1):
    return pl.pallas_call(
        _stage_a_body,
        grid=(NBLK,),
        in_specs=[
            pl.BlockSpec((NC, ROWBLK, 1), lambda i: (0, i, 0)),
            pl.BlockSpec((ROWBLK, D), lambda i: (i, 0)),
            pl.BlockSpec((D, D), lambda i: (0, 0)),
        ],
        out_specs=[
            pl.BlockSpec((ROWBLK, DH), lambda i: (i, 0)),
            pl.BlockSpec((ROWBLK, DH), lambda i: (i, 0)),
            pl.BlockSpec((ROWBLK, 1), lambda i: (i, 0)),
        ],
        out_shape=[
            jax.ShapeDtypeStruct((N, DH), jnp.float32),
            jax.ShapeDtypeStruct((N, DH), jnp.float32),
            jax.ShapeDtypeStruct((N, 1), jnp.float32),
        ],
    )(degp.reshape(NC, NPAD, 1), h0, W1)


def _stage_b_body(aggp_ref, glo_ref, ghi_ref, dinv_ref, w2_ref, b1_ref,
                  g2lo_ref, g2hi_ref):
    dinv = dinv_ref[...]
    g1 = jnp.concatenate([glo_ref[...], ghi_ref[...]], axis=1)
    agg = jnp.concatenate([aggp_ref[0], aggp_ref[1]], axis=1)
    h1 = jnp.maximum(dinv * (agg + g1) + b1_ref[...], 0.0)
    g2 = lax.dot_general(h1, w2_ref[...], (((1,), (1,)), ((), ())),
                         preferred_element_type=jnp.float32) * dinv
    g2lo_ref[...] = g2[:, :DH]
    g2hi_ref[...] = g2[:, DH:]


def _tc_stage_b(aggp, glo, ghi, dinv, W2, b1):
    return pl.pallas_call(
        _stage_b_body,
        grid=(NBLK,),
        in_specs=[
            pl.BlockSpec((NC, ROWBLK, DH), lambda i: (0, i, 0)),
            pl.BlockSpec((ROWBLK, DH), lambda i: (i, 0)),
            pl.BlockSpec((ROWBLK, DH), lambda i: (i, 0)),
            pl.BlockSpec((ROWBLK, 1), lambda i: (i, 0)),
            pl.BlockSpec((D, D), lambda i: (0, 0)),
            pl.BlockSpec((1, D), lambda i: (0, 0)),
        ],
        out_specs=[
            pl.BlockSpec((ROWBLK, DH), lambda i: (i, 0)),
            pl.BlockSpec((ROWBLK, DH), lambda i: (i, 0)),
        ],
        out_shape=[
            jax.ShapeDtypeStruct((N, DH), jnp.float32),
            jax.ShapeDtypeStruct((N, DH), jnp.float32),
        ],
    )(aggp, glo, ghi, dinv, W2, b1.reshape(1, D))


def _stage_c_body(aggp_ref, glo_ref, ghi_ref, dinv_ref, b2_ref, batch_ref,
                  wc_ref, bc_ref, out_ref, sums_ref, cnt_ref):
    i = pl.program_id(0)

    @pl.when(i == 0)
    def _init():
        sums_ref[...] = jnp.zeros_like(sums_ref)
        cnt_ref[...] = jnp.zeros_like(cnt_ref)

    dinv = dinv_ref[...]
    g2 = jnp.concatenate([glo_ref[...], ghi_ref[...]], axis=1)
    agg = jnp.concatenate([aggp_ref[0], aggp_ref[1]], axis=1)
    h2 = jnp.maximum(dinv * (agg + g2) + b2_ref[...], 0.0)
    oh = (batch_ref[...]
          == lax.broadcasted_iota(jnp.int32, (ROWBLK, G), 1)
          ).astype(jnp.float32)
    sums_ref[...] += lax.dot_general(oh, h2, (((0,), (0,)), ((), ())),
                                     preferred_element_type=jnp.float32)
    cnt_ref[...] += jnp.sum(oh, axis=0)[:, None]

    @pl.when(i == NBLK - 1)
    def _fin():
        pooled = sums_ref[...] / jnp.maximum(cnt_ref[...], 1.0)
        out_ref[...] = lax.dot_general(
            pooled, wc_ref[...], (((1,), (1,)), ((), ())),
            preferred_element_type=jnp.float32) + bc_ref[...]


def _tc_stage_c(aggp, glo, ghi, dinv, b2, batch, Wc, bc):
    return pl.pallas_call(
        _stage_c_body,
        grid=(NBLK,),
        in_specs=[
            pl.BlockSpec((NC, ROWBLK, DH), lambda i: (0, i, 0)),
            pl.BlockSpec((ROWBLK, DH), lambda i: (i, 0)),
            pl.BlockSpec((ROWBLK, DH), lambda i: (i, 0)),
            pl.BlockSpec((ROWBLK, 1), lambda i: (i, 0)),
            pl.BlockSpec((1, D), lambda i: (0, 0)),
            pl.BlockSpec((ROWBLK, 1), lambda i: (i, 0)),
            pl.BlockSpec((LOUT, D), lambda i: (0, 0)),
            pl.BlockSpec((1, LOUT), lambda i: (0, 0)),
        ],
        out_specs=pl.BlockSpec((G, LOUT), lambda i: (0, 0)),
        out_shape=jax.ShapeDtypeStruct((G, LOUT), jnp.float32),
        scratch_shapes=[
            pltpu.VMEM((G, D), jnp.float32),
            pltpu.VMEM((G, 1), jnp.float32),
        ],
    )(aggp, glo, ghi, dinv, b2.reshape(1, D), batch.reshape(N, 1), Wc,
      bc.reshape(1, LOUT))


def kernel(x, edge_index, batch, emb, W1, b1, W2, b2, Wc, bc):
    x1 = x.reshape(N)
    zeros1 = jnp.zeros((NPAD,), jnp.float32)
    zeros2 = jnp.zeros((NPAD, DH), jnp.float32)
    ones_c = jnp.ones((C,), jnp.float32)

    h0, degp = _sc_embed_deg(x1, emb, edge_index, zeros1, ones_c)
    glo, ghi, dinv = _tc_stage_a(degp.reshape(NC, NPAD), h0, W1)
    agg1 = _sc_edge_agg(edge_index, glo, ghi, zeros2)
    g2lo, g2hi = _tc_stage_b(agg1, glo, ghi, dinv, W2, b1)
    agg2 = _sc_edge_agg(edge_index, g2lo, g2hi, zeros2)
    return _tc_stage_c(agg2, g2lo, g2hi, dinv, b2, batch, Wc, bc)


# speedup vs baseline: 38.0128x; 3.1597x over previous
import functools

import jax
import jax.numpy as jnp
from jax import lax
from jax.experimental import pallas as pl
from jax.experimental.pallas import tpu as pltpu
from jax.experimental.pallas import tpu_sc as plsc

N = 100000
E = 3200000
D = 16
DH = D // 2
V = 50000
G = 128
LOUT = 10

NC = 2
NS = 16
NW = NC * NS

C = 128
NPAD = 100096
SLICE = NPAD // NS
NCH_E = E // C
NCH_N = N // C
NTAIL = N - NCH_N * C
ROWBLK = 800
NBLK = N // ROWBLK

_mesh = plsc.VectorSubcoreMesh(core_axis_name="c", subcore_axis_name="s")
_sc_params = pltpu.CompilerParams(use_tc_tiling_on_sc=False)


@functools.partial(
    pl.kernel,
    out_type=(
        jax.ShapeDtypeStruct((N, D), jnp.float32),
        jax.ShapeDtypeStruct((NC * NPAD,), jnp.float32),
    ),
    mesh=_mesh,
    scratch_types=[
        pltpu.VMEM((C,), jnp.int32),
        pltpu.VMEM((C, D), jnp.float32),
        pltpu.VMEM((NTAIL,), jnp.int32),
        pltpu.VMEM((NTAIL, D), jnp.float32),
        pltpu.VMEM((1, C), jnp.int32),
        pltpu.VMEM((C,), jnp.float32),
        pltpu.VMEM((SLICE,), jnp.float32),
        pltpu.VMEM_SHARED((NPAD,), jnp.float32),
        pltpu.SemaphoreType.DMA,
    ],
    compiler_params=_sc_params,
)
def _sc_embed_deg(x_hbm, emb_hbm, ei_hbm, zeros1_hbm, ones_hbm,
                  h0_hbm, degp_hbm,
                  xidx, rows, xidx_t, rows_t, didx, ones_v, zb, deg_sh, sem):
    c = lax.axis_index("c")
    s = lax.axis_index("s")
    wid = s * NC + c
    pltpu.sync_copy(zeros1_hbm.at[pl.ds(s * SLICE, SLICE)], zb)
    pltpu.sync_copy(zb, deg_sh.at[pl.ds(s * SLICE, SLICE)])
    pltpu.sync_copy(ones_hbm, ones_v)
    plsc.subcore_barrier()

    nte = 24 + (wid < NCH_N - 24 * NW).astype(jnp.int32)

    def ebody(t, carry):
        j = wid + NW * t
        base = j * C
        pltpu.sync_copy(x_hbm.at[pl.ds(base, C)], xidx)
        pltpu.async_copy(emb_hbm.at[xidx], rows, sem).wait()
        pltpu.sync_copy(rows, h0_hbm.at[pl.ds(base, C)])
        return carry

    lax.fori_loop(0, nte, ebody, 0)

    @pl.when(wid == 13)
    def _tail():
        pltpu.sync_copy(x_hbm.at[pl.ds(NCH_N * C, NTAIL)], xidx_t)
        pltpu.async_copy(emb_hbm.at[xidx_t], rows_t, sem).wait()
        pltpu.sync_copy(rows_t, h0_hbm.at[pl.ds(NCH_N * C, NTAIL)])

    ntd = 781 + (wid < NCH_E - 781 * NW).astype(jnp.int32)

    def dbody(t, carry):
        j = wid + NW * t
        base = j * C
        pltpu.sync_copy(ei_hbm.at[1, pl.ds(base, C)], didx.at[0])
        pltpu.sync_copy(ones_v, deg_sh.at[didx.at[0]], add=True)
        return carry

    lax.fori_loop(0, ntd, dbody, 0)
    plsc.subcore_barrier()
    pltpu.sync_copy(deg_sh.at[pl.ds(s * SLICE, SLICE)], zb)
    pltpu.sync_copy(zb, degp_hbm.at[pl.ds(c * NPAD + s * SLICE, SLICE)])


NB = 3
KB = 8
EBLK = KB * C
NBLKS = E // EBLK
NTB0 = NBLKS // NS
NTBR = NBLKS - NTB0 * NS


@functools.partial(
    pl.kernel,
    out_type=jax.ShapeDtypeStruct((NC, NPAD, DH), jnp.float32),
    mesh=_mesh,
    scratch_types=[
        pltpu.VMEM((NB, KB, C), jnp.int32),
        pltpu.VMEM((NB, KB, C), jnp.int32),
        pltpu.VMEM((NB, KB, C, DH), jnp.float32),
        pltpu.VMEM((SLICE, DH), jnp.float32),
        pltpu.VMEM_SHARED((NPAD, DH), jnp.float32),
        pltpu.SemaphoreType.DMA,
        pltpu.SemaphoreType.DMA,
        pltpu.SemaphoreType.DMA,
    ],
    compiler_params=_sc_params,
)
def _sc_edge_agg(ei3_hbm, glo_hbm, ghi_hbm, zeros2_hbm, aggp_hbm,
                 sidx3, didx3, rows3, zb, agg_sh, sem_i, sem_g, sem_s):
    c = lax.axis_index("c")
    s = lax.axis_index("s")
    pltpu.sync_copy(zeros2_hbm.at[pl.ds(s * SLICE, SLICE)], zb)
    pltpu.sync_copy(zb, agg_sh.at[pl.ds(s * SLICE, SLICE)])
    plsc.subcore_barrier()

    nt = NTB0 + (s < NTBR).astype(jnp.int32)

    def idx_start(t, buf):
        j8 = jnp.minimum(s + NS * t, NBLKS - 1) * KB
        pltpu.make_async_copy(ei3_hbm.at[0, pl.ds(j8, KB)],
                              sidx3.at[buf], sem_i).start()
        pltpu.make_async_copy(ei3_hbm.at[1, pl.ds(j8, KB)],
                              didx3.at[buf], sem_i).start()

    def idx_wait(buf):
        pltpu.make_async_copy(ei3_hbm.at[0, pl.ds(0, KB)],
                              sidx3.at[buf], sem_i).wait()
        pltpu.make_async_copy(ei3_hbm.at[1, pl.ds(0, KB)],
                              didx3.at[buf], sem_i).wait()

    def scat_wait(buf):
        for k in range(KB):
            pltpu.make_async_copy(rows3.at[buf, k],
                                  agg_sh.at[didx3.at[buf, k]], sem_s).wait()

    def sweep(g_hbm):
        idx_start(0, 0)

        def body(t, carry):
            bb = lax.rem(t, NB)
            bn = lax.rem(t + 1, NB)
            @pl.when(t >= 2)
            def _drain():
                scat_wait(bn)

            idx_start(t + 1, bn)
            idx_wait(bb)
            for k in range(KB):
                pltpu.make_async_copy(g_hbm.at[sidx3.at[bb, k]],
                                      rows3.at[bb, k], sem_g).start()
            for k in range(KB):
                pltpu.make_async_copy(g_hbm.at[sidx3.at[bb, k]],
                                      rows3.at[bb, k], sem_g).wait()
            for k in range(KB):
                pltpu.make_async_copy(rows3.at[bb, k],
                                      agg_sh.at[didx3.at[bb, k]],
                                      sem_s).start(add=True)
            return carry

        lax.fori_loop(0, nt, body, 0)

    @pl.when(c == 0)
    def _lo():
        sweep(glo_hbm)

    @pl.when(c == 1)
    def _hi():
        sweep(ghi_hbm)

    for _ in range(2):
        scat_wait(0)
    idx_wait(0)

    plsc.subcore_barrier()
    pltpu.sync_copy(agg_sh.at[pl.ds(s * SLICE, SLICE)], zb)
    pltpu.sync_copy(zb, aggp_hbm.at[c, pl.ds(s * SLICE, SLICE)])


def _stage_a_body(degp_ref, h0_ref, w1_ref, glo_ref, ghi_ref, dinv_ref):
    deg = degp_ref[0] + degp_ref[1] + 1.0
    dinv = lax.rsqrt(deg)
    g1 = lax.dot_general(h0_ref[...], w1_ref[...],
                         (((1,), (1,)), ((), ())),
                         preferred_element_type=jnp.float32) * dinv
    glo_ref[...] = g1[:, :DH]
    ghi_ref[...] = g1[:, DH:]
    dinv_ref[...] = dinv


def _tc_stage_a(degp, h0, W1):
    return pl.pallas_call(
        _stage_a_body,
        grid=(NBLK,),
        in_specs=[
            pl.BlockSpec((NC, ROWBLK, 1), lambda i: (0, i, 0)),
            pl.BlockSpec((ROWBLK, D), lambda i: (i, 0)),
            pl.BlockSpec((D, D), lambda i: (0, 0)),
        ],
        out_specs=[
            pl.BlockSpec((ROWBLK, DH), lambda i: (i, 0)),
            pl.BlockSpec((ROWBLK, DH), lambda i: (i, 0)),
            pl.BlockSpec((ROWBLK, 1), lambda i: (i, 0)),
        ],
        out_shape=[
            jax.ShapeDtypeStruct((N, DH), jnp.float32),
            jax.ShapeDtypeStruct((N, DH), jnp.float32),
            jax.ShapeDtypeStruct((N, 1), jnp.float32),
        ],
    )(degp.reshape(NC, NPAD, 1), h0, W1)


def _stage_b_body(aggp_ref, glo_ref, ghi_ref, dinv_ref, w2_ref, b1_ref,
                  g2lo_ref, g2hi_ref):
    dinv = dinv_ref[...]
    g1 = jnp.concatenate([glo_ref[...], ghi_ref[...]], axis=1)
    agg = jnp.concatenate([aggp_ref[0], aggp_ref[1]], axis=1)
    h1 = jnp.maximum(dinv * (agg + g1) + b1_ref[...], 0.0)
    g2 = lax.dot_general(h1, w2_ref[...], (((1,), (1,)), ((), ())),
                         preferred_element_type=jnp.float32) * dinv
    g2lo_ref[...] = g2[:, :DH]
    g2hi_ref[...] = g2[:, DH:]


def _tc_stage_b(aggp, glo, ghi, dinv, W2, b1):
    return pl.pallas_call(
        _stage_b_body,
        grid=(NBLK,),
        in_specs=[
            pl.BlockSpec((NC, ROWBLK, DH), lambda i: (0, i, 0)),
            pl.BlockSpec((ROWBLK, DH), lambda i: (i, 0)),
            pl.BlockSpec((ROWBLK, DH), lambda i: (i, 0)),
            pl.BlockSpec((ROWBLK, 1), lambda i: (i, 0)),
            pl.BlockSpec((D, D), lambda i: (0, 0)),
            pl.BlockSpec((1, D), lambda i: (0, 0)),
        ],
        out_specs=[
            pl.BlockSpec((ROWBLK, DH), lambda i: (i, 0)),
            pl.BlockSpec((ROWBLK, DH), lambda i: (i, 0)),
        ],
        out_shape=[
            jax.ShapeDtypeStruct((N, DH), jnp.float32),
            jax.ShapeDtypeStruct((N, DH), jnp.float32),
        ],
    )(aggp, glo, ghi, dinv, W2, b1.reshape(1, D))


def _stage_c_body(aggp_ref, glo_ref, ghi_ref, dinv_ref, b2_ref, batch_ref,
                  wc_ref, bc_ref, out_ref, sums_ref, cnt_ref):
    i = pl.program_id(0)

    @pl.when(i == 0)
    def _init():
        sums_ref[...] = jnp.zeros_like(sums_ref)
        cnt_ref[...] = jnp.zeros_like(cnt_ref)

    dinv = dinv_ref[...]
    g2 = jnp.concatenate([glo_ref[...], ghi_ref[...]], axis=1)
    agg = jnp.concatenate([aggp_ref[0], aggp_ref[1]], axis=1)
    h2 = jnp.maximum(dinv * (agg + g2) + b2_ref[...], 0.0)
    oh = (batch_ref[...]
          == lax.broadcasted_iota(jnp.int32, (ROWBLK, G), 1)
          ).astype(jnp.float32)
    sums_ref[...] += lax.dot_general(oh, h2, (((0,), (0,)), ((), ())),
                                     preferred_element_type=jnp.float32)
    cnt_ref[...] += jnp.sum(oh, axis=0)[:, None]

    @pl.when(i == NBLK - 1)
    def _fin():
        pooled = sums_ref[...] / jnp.maximum(cnt_ref[...], 1.0)
        out_ref[...] = lax.dot_general(
            pooled, wc_ref[...], (((1,), (1,)), ((), ())),
            preferred_element_type=jnp.float32) + bc_ref[...]


def _tc_stage_c(aggp, glo, ghi, dinv, b2, batch, Wc, bc):
    return pl.pallas_call(
        _stage_c_body,
        grid=(NBLK,),
        in_specs=[
            pl.BlockSpec((NC, ROWBLK, DH), lambda i: (0, i, 0)),
            pl.BlockSpec((ROWBLK, DH), lambda i: (i, 0)),
            pl.BlockSpec((ROWBLK, DH), lambda i: (i, 0)),
            pl.BlockSpec((ROWBLK, 1), lambda i: (i, 0)),
            pl.BlockSpec((1, D), lambda i: (0, 0)),
            pl.BlockSpec((ROWBLK, 1), lambda i: (i, 0)),
            pl.BlockSpec((LOUT, D), lambda i: (0, 0)),
            pl.BlockSpec((1, LOUT), lambda i: (0, 0)),
        ],
        out_specs=pl.BlockSpec((G, LOUT), lambda i: (0, 0)),
        out_shape=jax.ShapeDtypeStruct((G, LOUT), jnp.float32),
        scratch_shapes=[
            pltpu.VMEM((G, D), jnp.float32),
            pltpu.VMEM((G, 1), jnp.float32),
        ],
    )(aggp, glo, ghi, dinv, b2.reshape(1, D), batch.reshape(N, 1), Wc,
      bc.reshape(1, LOUT))


def kernel(x, edge_index, batch, emb, W1, b1, W2, b2, Wc, bc):
    x1 = x.reshape(N)
    zeros1 = jnp.zeros((NPAD,), jnp.float32)
    zeros2 = jnp.zeros((NPAD, DH), jnp.float32)
    ones_c = jnp.ones((C,), jnp.float32)

    ei3 = edge_index.reshape(2, NCH_E, C)
    h0, degp = _sc_embed_deg(x1, emb, edge_index, zeros1, ones_c)
    glo, ghi, dinv = _tc_stage_a(degp.reshape(NC, NPAD), h0, W1)
    agg1 = _sc_edge_agg(ei3, glo, ghi, zeros2)
    g2lo, g2hi = _tc_stage_b(agg1, glo, ghi, dinv, W2, b1)
    agg2 = _sc_edge_agg(ei3, g2lo, g2hi, zeros2)
    return _tc_stage_c(agg2, g2lo, g2hi, dinv, b2, batch, Wc, bc)


# speedup vs baseline: 47.1304x; 1.2399x over previous
import functools

import jax
import jax.numpy as jnp
from jax import lax
from jax.experimental import pallas as pl
from jax.experimental.pallas import tpu as pltpu
from jax.experimental.pallas import tpu_sc as plsc

N = 100000
E = 3200000
D = 16
DH = D // 2
V = 50000
G = 128
LOUT = 10

NC = 2
NS = 16
NW = NC * NS

C = 128
NPAD = 100096
SLICE = NPAD // NS
NCH_E = E // C
NCH_N = N // C
NTAIL = N - NCH_N * C
ROWBLK = 800
NBLK = N // ROWBLK

_mesh = plsc.VectorSubcoreMesh(core_axis_name="c", subcore_axis_name="s")
_sc_params = pltpu.CompilerParams(use_tc_tiling_on_sc=False)


NB = 3
KB = 8
EBLK = KB * C
NBLKS = E // EBLK
XR = 100
XK = 8
XBLKS = N // (XR * XK)


@functools.partial(
    pl.kernel,
    out_type=(
        jax.ShapeDtypeStruct((N, D), jnp.float32),
        jax.ShapeDtypeStruct((NC * NPAD,), jnp.float32),
    ),
    mesh=_mesh,
    scratch_types=[
        pltpu.VMEM((NB, XK, XR), jnp.int32),
        pltpu.VMEM((NB, XK * XR, D), jnp.float32),
        pltpu.VMEM((NB, KB, C), jnp.int32),
        pltpu.VMEM((C,), jnp.float32),
        pltpu.VMEM((SLICE,), jnp.float32),
        pltpu.VMEM_SHARED((NPAD,), jnp.float32),
        pltpu.SemaphoreType.DMA,
        pltpu.SemaphoreType.DMA,
        pltpu.SemaphoreType.DMA,
    ],
    compiler_params=_sc_params,
)
def _sc_embed_deg(x3_hbm, emb_hbm, ei3_hbm, zeros1_hbm, ones_hbm,
                  h0_hbm, degp_hbm,
                  xidx3, erows3, didx3, ones_v, zb, deg_sh,
                  sem_i, sem_g, sem_s):
    c = lax.axis_index("c")
    s = lax.axis_index("s")
    wid = s * NC + c
    pltpu.sync_copy(zeros1_hbm.at[pl.ds(s * SLICE, SLICE)], zb)
    pltpu.sync_copy(zb, deg_sh.at[pl.ds(s * SLICE, SLICE)])
    pltpu.sync_copy(ones_hbm, ones_v)
    plsc.subcore_barrier()

    nte = (XBLKS // NW) + (wid < XBLKS - (XBLKS // NW) * NW).astype(jnp.int32)

    def xidx_start(t, buf):
        j = jnp.minimum(wid + NW * t, XBLKS - 1)
        pltpu.make_async_copy(x3_hbm.at[j], xidx3.at[buf], sem_i).start()

    def xidx_wait(buf):
        pltpu.make_async_copy(x3_hbm.at[0], xidx3.at[buf], sem_i).wait()

    def h0_wait(buf):
        pltpu.make_async_copy(erows3.at[buf],
                              h0_hbm.at[pl.ds(0, XK * XR)], sem_s).wait()

    xidx_start(0, 0)

    def ebody(t, carry):
        bb = lax.rem(t, NB)
        bn = lax.rem(t + 1, NB)

        @pl.when(t >= 2)
        def _drain():
            h0_wait(bn)

        xidx_start(t + 1, bn)
        xidx_wait(bb)
        for k in range(XK):
            pltpu.make_async_copy(emb_hbm.at[xidx3.at[bb, k]],
                                  erows3.at[bb, pl.ds(k * XR, XR)],
                                  sem_g).start()
        for k in range(XK):
            pltpu.make_async_copy(emb_hbm.at[xidx3.at[bb, k]],
                                  erows3.at[bb, pl.ds(k * XR, XR)],
                                  sem_g).wait()
        j = wid + NW * t
        pltpu.make_async_copy(erows3.at[bb],
                              h0_hbm.at[pl.ds(j * (XK * XR), XK * XR)],
                              sem_s).start()
        return carry

    lax.fori_loop(0, nte, ebody, 0)
    for _ in range(2):
        h0_wait(0)
    xidx_wait(0)

    ntd = (NBLKS // NW) + (wid < NBLKS - (NBLKS // NW) * NW).astype(jnp.int32)

    def didx_start(t, buf):
        j8 = jnp.minimum(wid + NW * t, NBLKS - 1) * KB
        pltpu.make_async_copy(ei3_hbm.at[1, pl.ds(j8, KB)],
                              didx3.at[buf], sem_i).start()

    def didx_wait(buf):
        pltpu.make_async_copy(ei3_hbm.at[1, pl.ds(0, KB)],
                              didx3.at[buf], sem_i).wait()

    def deg_scat_wait(buf):
        for k in range(KB):
            pltpu.make_async_copy(ones_v, deg_sh.at[didx3.at[buf, k]],
                                  sem_s).wait()

    didx_start(0, 0)

    def dbody(t, carry):
        bb = lax.rem(t, NB)
        bn = lax.rem(t + 1, NB)

        @pl.when(t >= 2)
        def _drain():
            deg_scat_wait(bn)

        didx_start(t + 1, bn)
        didx_wait(bb)
        for k in range(KB):
            pltpu.make_async_copy(ones_v, deg_sh.at[didx3.at[bb, k]],
                                  sem_s).start(add=True)
        return carry

    lax.fori_loop(0, ntd, dbody, 0)
    for _ in range(2):
        deg_scat_wait(0)
    didx_wait(0)

    plsc.subcore_barrier()
    pltpu.sync_copy(deg_sh.at[pl.ds(s * SLICE, SLICE)], zb)
    pltpu.sync_copy(zb, degp_hbm.at[pl.ds(c * NPAD + s * SLICE, SLICE)])


NTB0 = NBLKS // NS
NTBR = NBLKS - NTB0 * NS


@functools.partial(
    pl.kernel,
    out_type=jax.ShapeDtypeStruct((NC, NPAD, DH), jnp.float32),
    mesh=_mesh,
    scratch_types=[
        pltpu.VMEM((NB, KB, C), jnp.int32),
        pltpu.VMEM((NB, KB, C), jnp.int32),
        pltpu.VMEM((NB, KB, C, DH), jnp.float32),
        pltpu.VMEM((SLICE, DH), jnp.float32),
        pltpu.VMEM_SHARED((NPAD, DH), jnp.float32),
        pltpu.SemaphoreType.DMA,
        pltpu.SemaphoreType.DMA,
        pltpu.SemaphoreType.DMA,
    ],
    compiler_params=_sc_params,
)
def _sc_edge_agg(ei3_hbm, glo_hbm, ghi_hbm, zeros2_hbm, aggp_hbm,
                 sidx3, didx3, rows3, zb, agg_sh, sem_i, sem_g, sem_s):
    c = lax.axis_index("c")
    s = lax.axis_index("s")
    pltpu.sync_copy(zeros2_hbm.at[pl.ds(s * SLICE, SLICE)], zb)
    pltpu.sync_copy(zb, agg_sh.at[pl.ds(s * SLICE, SLICE)])
    plsc.subcore_barrier()

    nt = NTB0 + (s < NTBR).astype(jnp.int32)

    def idx_start(t, buf):
        j8 = jnp.minimum(s + NS * t, NBLKS - 1) * KB
        pltpu.make_async_copy(ei3_hbm.at[0, pl.ds(j8, KB)],
                              sidx3.at[buf], sem_i).start()
        pltpu.make_async_copy(ei3_hbm.at[1, pl.ds(j8, KB)],
                              didx3.at[buf], sem_i).start()

    def idx_wait(buf):
        pltpu.make_async_copy(ei3_hbm.at[0, pl.ds(0, KB)],
                              sidx3.at[buf], sem_i).wait()
        pltpu.make_async_copy(ei3_hbm.at[1, pl.ds(0, KB)],
                              didx3.at[buf], sem_i).wait()

    def scat_wait(buf):
        for k in range(KB):
            pltpu.make_async_copy(rows3.at[buf, k],
                                  agg_sh.at[didx3.at[buf, k]], sem_s).wait()

    def sweep(g_hbm):
        idx_start(0, 0)

        def body(t, carry):
            bb = lax.rem(t, NB)
            bn = lax.rem(t + 1, NB)
            @pl.when(t >= 2)
            def _drain():
                scat_wait(bn)

            idx_start(t + 1, bn)
            idx_wait(bb)
            for k in range(KB):
                pltpu.make_async_copy(g_hbm.at[sidx3.at[bb, k]],
                                      rows3.at[bb, k], sem_g).start()
            for k in range(KB):
                pltpu.make_async_copy(g_hbm.at[sidx3.at[bb, k]],
                                      rows3.at[bb, k], sem_g).wait()
            for k in range(KB):
                pltpu.make_async_copy(rows3.at[bb, k],
                                      agg_sh.at[didx3.at[bb, k]],
                                      sem_s).start(add=True)
            return carry

        lax.fori_loop(0, nt, body, 0)

    @pl.when(c == 0)
    def _lo():
        sweep(glo_hbm)

    @pl.when(c == 1)
    def _hi():
        sweep(ghi_hbm)

    for _ in range(2):
        scat_wait(0)
    idx_wait(0)

    plsc.subcore_barrier()
    pltpu.sync_copy(agg_sh.at[pl.ds(s * SLICE, SLICE)], zb)
    pltpu.sync_copy(zb, aggp_hbm.at[c, pl.ds(s * SLICE, SLICE)])


def _stage_a_body(degp_ref, h0_ref, w1_ref, glo_ref, ghi_ref, dinv_ref):
    deg = degp_ref[0] + degp_ref[1] + 1.0
    dinv = lax.rsqrt(deg)
    g1 = lax.dot_general(h0_ref[...], w1_ref[...],
                         (((1,), (1,)), ((), ())),
                         preferred_element_type=jnp.float32) * dinv
    glo_ref[...] = g1[:, :DH]
    ghi_ref[...] = g1[:, DH:]
    dinv_ref[...] = dinv


def _tc_stage_a(degp, h0, W1):
    return pl.pallas_call(
        _stage_a_body,
        grid=(NBLK,),
        in_specs=[
            pl.BlockSpec((NC, ROWBLK, 1), lambda i: (0, i, 0)),
            pl.BlockSpec((ROWBLK, D), lambda i: (i, 0)),
            pl.BlockSpec((D, D), lambda i: (0, 0)),
        ],
        out_specs=[
            pl.BlockSpec((ROWBLK, DH), lambda i: (i, 0)),
            pl.BlockSpec((ROWBLK, DH), lambda i: (i, 0)),
            pl.BlockSpec((ROWBLK, 1), lambda i: (i, 0)),
        ],
        out_shape=[
            jax.ShapeDtypeStruct((N, DH), jnp.float32),
            jax.ShapeDtypeStruct((N, DH), jnp.float32),
            jax.ShapeDtypeStruct((N, 1), jnp.float32),
        ],
    )(degp.reshape(NC, NPAD, 1), h0, W1)


def _stage_b_body(aggp_ref, glo_ref, ghi_ref, dinv_ref, w2_ref, b1_ref,
                  g2lo_ref, g2hi_ref):
    dinv = dinv_ref[...]
    g1 = jnp.concatenate([glo_ref[...], ghi_ref[...]], axis=1)
    agg = jnp.concatenate([aggp_ref[0], aggp_ref[1]], axis=1)
    h1 = jnp.maximum(dinv * (agg + g1) + b1_ref[...], 0.0)
    g2 = lax.dot_general(h1, w2_ref[...], (((1,), (1,)), ((), ())),
                         preferred_element_type=jnp.float32) * dinv
    g2lo_ref[...] = g2[:, :DH]
    g2hi_ref[...] = g2[:, DH:]


def _tc_stage_b(aggp, glo, ghi, dinv, W2, b1):
    return pl.pallas_call(
        _stage_b_body,
        grid=(NBLK,),
        in_specs=[
            pl.BlockSpec((NC, ROWBLK, DH), lambda i: (0, i, 0)),
            pl.BlockSpec((ROWBLK, DH), lambda i: (i, 0)),
            pl.BlockSpec((ROWBLK, DH), lambda i: (i, 0)),
            pl.BlockSpec((ROWBLK, 1), lambda i: (i, 0)),
            pl.BlockSpec((D, D), lambda i: (0, 0)),
            pl.BlockSpec((1, D), lambda i: (0, 0)),
        ],
        out_specs=[
            pl.BlockSpec((ROWBLK, DH), lambda i: (i, 0)),
            pl.BlockSpec((ROWBLK, DH), lambda i: (i, 0)),
        ],
        out_shape=[
            jax.ShapeDtypeStruct((N, DH), jnp.float32),
            jax.ShapeDtypeStruct((N, DH), jnp.float32),
        ],
    )(aggp, glo, ghi, dinv, W2, b1.reshape(1, D))


def _stage_c_body(aggp_ref, glo_ref, ghi_ref, dinv_ref, b2_ref, batch_ref,
                  wc_ref, bc_ref, out_ref, sums_ref, cnt_ref):
    i = pl.program_id(0)

    @pl.when(i == 0)
    def _init():
        sums_ref[...] = jnp.zeros_like(sums_ref)
        cnt_ref[...] = jnp.zeros_like(cnt_ref)

    dinv = dinv_ref[...]
    g2 = jnp.concatenate([glo_ref[...], ghi_ref[...]], axis=1)
    agg = jnp.concatenate([aggp_ref[0], aggp_ref[1]], axis=1)
    h2 = jnp.maximum(dinv * (agg + g2) + b2_ref[...], 0.0)
    oh = (batch_ref[...]
          == lax.broadcasted_iota(jnp.int32, (ROWBLK, G), 1)
          ).astype(jnp.float32)
    sums_ref[...] += lax.dot_general(oh, h2, (((0,), (0,)), ((), ())),
                                     preferred_element_type=jnp.float32)
    cnt_ref[...] += jnp.sum(oh, axis=0)[:, None]

    @pl.when(i == NBLK - 1)
    def _fin():
        pooled = sums_ref[...] / jnp.maximum(cnt_ref[...], 1.0)
        out_ref[...] = lax.dot_general(
            pooled, wc_ref[...], (((1,), (1,)), ((), ())),
            preferred_element_type=jnp.float32) + bc_ref[...]


def _tc_stage_c(aggp, glo, ghi, dinv, b2, batch, Wc, bc):
    return pl.pallas_call(
        _stage_c_body,
        grid=(NBLK,),
        in_specs=[
            pl.BlockSpec((NC, ROWBLK, DH), lambda i: (0, i, 0)),
            pl.BlockSpec((ROWBLK, DH), lambda i: (i, 0)),
            pl.BlockSpec((ROWBLK, DH), lambda i: (i, 0)),
            pl.BlockSpec((ROWBLK, 1), lambda i: (i, 0)),
            pl.BlockSpec((1, D), lambda i: (0, 0)),
            pl.BlockSpec((ROWBLK, 1), lambda i: (i, 0)),
            pl.BlockSpec((LOUT, D), lambda i: (0, 0)),
            pl.BlockSpec((1, LOUT), lambda i: (0, 0)),
        ],
        out_specs=pl.BlockSpec((G, LOUT), lambda i: (0, 0)),
        out_shape=jax.ShapeDtypeStruct((G, LOUT), jnp.float32),
        scratch_shapes=[
            pltpu.VMEM((G, D), jnp.float32),
            pltpu.VMEM((G, 1), jnp.float32),
        ],
    )(aggp, glo, ghi, dinv, b2.reshape(1, D), batch.reshape(N, 1), Wc,
      bc.reshape(1, LOUT))


def kernel(x, edge_index, batch, emb, W1, b1, W2, b2, Wc, bc):
    x3 = x.reshape(XBLKS, XK, XR)
    zeros1 = jnp.zeros((NPAD,), jnp.float32)
    zeros2 = jnp.zeros((NPAD, DH), jnp.float32)
    ones_c = jnp.ones((C,), jnp.float32)

    ei3 = edge_index.reshape(2, NCH_E, C)
    h0, degp = _sc_embed_deg(x3, emb, ei3, zeros1, ones_c)
    glo, ghi, dinv = _tc_stage_a(degp.reshape(NC, NPAD), h0, W1)
    agg1 = _sc_edge_agg(ei3, glo, ghi, zeros2)
    g2lo, g2hi = _tc_stage_b(agg1, glo, ghi, dinv, W2, b1)
    agg2 = _sc_edge_agg(ei3, g2lo, g2hi, zeros2)
    return _tc_stage_c(agg2, g2lo, g2hi, dinv, b2, batch, Wc, bc)
